# Initial kernel scaffold; baseline (speedup 1.0000x reference)
import jax
import jax.numpy as jnp
from jax.experimental import pallas as pl


def kernel(child_feats, edge_indices, edge_type_onehot, lengths, W1, b1, W2, b2, We0, be0, We1, be1, Ws, bs):
    raise NotImplementedError("write your pallas kernel here")


# speedup vs baseline: 8.8080x; 8.8080x over previous
import jax
import jax.numpy as jnp
from jax import lax
from jax.experimental import pallas as pl
from jax.experimental.pallas import tpu as pltpu
from jax.experimental.pallas import tpu_sc as plsc

_NC = 2
_NS = 16
_K = 80


def _leaky(x):
    return jnp.where(x >= 0, x, 0.1 * x)


def _dot(a, b):
    return jnp.dot(a, b, preferred_element_type=jnp.float32)


def kernel(child_feats, edge_indices, edge_type_onehot, lengths,
           W1, b1, W2, b2, We0, be0, We1, be1, Ws, bs):
    N = child_feats.shape[1]
    FEAT = child_feats.shape[2]
    E = edge_indices.shape[1]
    H = W2.shape[0]
    ET = edge_type_onehot.shape[2]

    x = child_feats[0]
    W1p = jnp.zeros((FEAT, H), jnp.float32).at[: W1.shape[0]].set(W1)
    ef = edge_indices[0, :, 0]
    et = edge_indices[0, :, 1]
    ety = jnp.argmax(edge_type_onehot[0], axis=1).astype(jnp.int32)
    cidx = ety * N + et
    NP = ((N + 8 * _NS - 1) // (8 * _NS)) * (8 * _NS)
    zrows = jnp.zeros((NP, H), jnp.float32)


    R = 1000
    grid = (N // R,)
    full = lambda shape: pl.BlockSpec(shape, lambda i: tuple(0 for _ in shape))
    rows2 = pl.BlockSpec((R, H), lambda i: (i, 0))

    def mlp_body(x_ref, w1_ref, b1_ref, w2_ref, b2_ref, wsrc_ref, wdst_ref,
                 t_ref, cf_ref, a_ref, c_ref):
        net = _leaky(_leaky(_dot(x_ref[...], w1_ref[...]) + b1_ref[...]))
        cf = _leaky(_dot(net, w2_ref[...]) + b2_ref[...])
        cf_ref[...] = cf
        a_ref[...] = _dot(cf, wsrc_ref[...])
        c_ref[...] = _dot(cf, wdst_ref[...])[None, :, :] + t_ref[...][:, None, :]

    cf0, A0, C0 = pl.pallas_call(
        mlp_body,
        grid=grid,
        in_specs=[
            pl.BlockSpec((R, FEAT), lambda i: (i, 0)),
            full((FEAT, H)), full((1, H)), full((H, H)), full((1, H)),
            full((H, H)), full((H, H)), full((ET, H)),
        ],
        out_specs=[rows2, rows2, pl.BlockSpec((ET, R, H), lambda i: (0, i, 0))],
        out_shape=[
            jax.ShapeDtypeStruct((N, H), jnp.float32),
            jax.ShapeDtypeStruct((N, H), jnp.float32),
            jax.ShapeDtypeStruct((ET, N, H), jnp.float32),
        ],
    )(x, W1p, b1.reshape(1, H), W2, b2.reshape(1, H),
      We0[:H], We0[H:2 * H], We0[2 * H:] + be0[None, :])

    epw = E // (_NC * _NS)
    nchunks = epw // _K
    rows_per_tile = NP // _NS

    assert epw % _K == 0 and nchunks % 2 == 1 and nchunks >= 3

    def sc_body(ef_h, ci_h, a_h, c_h, z_h, out_h,
                ie0, ic0, ie1, ic1, se0, se1, ar0, cr0, ar1, cr1, acc,
                sie0, sic0, sie1, sic1, sga0, sgc0, sga1, sgc1, ssc0, ssc1):
        cx = lax.axis_index("c")
        sx = lax.axis_index("s")
        wid = sx * _NC + cx
        rsl = pl.ds(sx * rows_per_tile, rows_per_tile)
        pltpu.sync_copy(z_h.at[rsl], acc.at[rsl])
        plsc.subcore_barrier()
        ebase = wid * epw

        slots = [(ie0, ic0, ar0, cr0, sie0, sic0, sga0, sgc0, se0, ssc0),
                 (ie1, ic1, ar1, cr1, sie1, sic1, sga1, sgc1, se1, ssc1)]

        def fire_idx(cid, p):
            B = slots[p]
            base = jnp.minimum(ebase + cid * _K, E - _K)
            pltpu.async_copy(ef_h.at[pl.ds(base, _K)], B[0], B[4])
            pltpu.async_copy(ci_h.at[pl.ds(base, _K)], B[1], B[5])

        def wait_idx(p):
            B = slots[p]
            pltpu.make_async_copy(ef_h.at[pl.ds(0, _K)], B[0], B[4]).wait()
            pltpu.make_async_copy(ci_h.at[pl.ds(0, _K)], B[1], B[5]).wait()

        def fire_gather(p):
            B = slots[p]
            pltpu.async_copy(a_h.at[B[0]], B[2], B[6])
            pltpu.async_copy(c_h.at[B[1]], B[3], B[7])

        def wait_gather(p):
            B = slots[p]
            pltpu.make_async_copy(a_h.at[B[0]], B[2], B[6]).wait()
            pltpu.make_async_copy(c_h.at[B[1]], B[3], B[7]).wait()

        def compute(p):
            B = slots[p]
            ie, ar, cr, se = B[0], B[2], B[3], B[8]

            def vec(e, carry2):
                for g in range(H // 16):
                    sl = pl.ds(16 * g, 16)
                    ar[e, sl] = jnp.maximum(ar[e, sl] + cr[e, sl], 0.0)
                return carry2

            lax.fori_loop(0, _K, vec, 0)
            for j in range(_K // 16):
                sl = pl.ds(16 * j, 16)
                se[sl] = ie[sl]

        def fire_scat(p):
            B = slots[p]
            pltpu.async_copy(B[2], acc.at[B[8]], B[9], add=True)

        def wait_scat(p):
            B = slots[p]
            pltpu.make_async_copy(B[2], acc.at[B[8]], B[9]).wait()

        def step(cid, p, drain_scat=True):
            wait_idx(1 - p)
            if drain_scat:
                wait_scat(1 - p)
            fire_gather(1 - p)
            wait_gather(p)
            compute(p)
            fire_scat(p)
            fire_idx(cid + 2, p)

        fire_idx(0, 0)
        wait_idx(0)
        fire_gather(0)
        fire_idx(1, 1)
        step(0, 0, drain_scat=False)

        def outer(i, carry):
            step(2 * i + 1, 1)
            step(2 * i + 2, 0)
            return carry

        lax.fori_loop(0, (nchunks - 1) // 2, outer, 0)
        wait_scat((nchunks - 1) % 2)
        wait_gather(nchunks % 2)
        wait_idx((nchunks + 1) % 2)
        plsc.subcore_barrier()
        pltpu.sync_copy(acc.at[rsl], out_h.at[cx, rsl])

    sc_call = pl.kernel(
        sc_body,
        out_type=jax.ShapeDtypeStruct((_NC, NP, H), jnp.float32),
        mesh=plsc.VectorSubcoreMesh(core_axis_name="c", subcore_axis_name="s",
                                    num_cores=_NC, num_subcores=_NS),
        scratch_types=(
            [pltpu.VMEM((_K,), jnp.int32)] * 6
            + [pltpu.VMEM((_K, H), jnp.float32)] * 4
            + [pltpu.VMEM_SHARED((NP, H), jnp.float32)]
            + [pltpu.SemaphoreType.DMA] * 10
        ),
    )

    P0 = sc_call(ef, cidx, A0, C0.reshape(ET * N, H), zrows)

    def mid_body(p_ref, wsrc_ref, wdst_ref, t_ref, cf_ref, a_ref, c_ref):
        cf = p_ref[0] + p_ref[1]
        cf_ref[...] = cf
        a_ref[...] = _dot(cf, wsrc_ref[...])
        c_ref[...] = _dot(cf, wdst_ref[...])[None, :, :] + t_ref[...][:, None, :]

    cf1, A1, C1 = pl.pallas_call(
        mid_body,
        grid=grid,
        in_specs=[
            pl.BlockSpec((_NC, R, H), lambda i: (0, i, 0)),
            full((H, H)), full((H, H)), full((ET, H)),
        ],
        out_specs=[rows2, rows2, pl.BlockSpec((ET, R, H), lambda i: (0, i, 0))],
        out_shape=[
            jax.ShapeDtypeStruct((N, H), jnp.float32),
            jax.ShapeDtypeStruct((N, H), jnp.float32),
            jax.ShapeDtypeStruct((ET, N, H), jnp.float32),
        ],
    )(P0, We1[:H], We1[H:2 * H], We1[2 * H:] + be1[None, :])

    P1 = sc_call(ef, cidx, A1, C1.reshape(ET * N, H), zrows)

    def out_body(p_ref, cf0_ref, cf1_ref, w0_ref, w1_ref, w2_ref, bs_ref, o_ref):
        cf2 = p_ref[0] + p_ref[1]
        o_ref[...] = _leaky(
            _dot(cf0_ref[...], w0_ref[...]) + _dot(cf1_ref[...], w1_ref[...])
            + _dot(cf2, w2_ref[...]) + bs_ref[...])

    out = pl.pallas_call(
        out_body,
        grid=grid,
        in_specs=[
            pl.BlockSpec((_NC, R, H), lambda i: (0, i, 0)),
            rows2, rows2,
            full((H, FEAT)), full((H, FEAT)), full((H, FEAT)), full((1, FEAT)),
        ],
        out_specs=pl.BlockSpec((R, FEAT), lambda i: (i, 0)),
        out_shape=jax.ShapeDtypeStruct((N, FEAT), jnp.float32),
    )(P1, cf0, cf1, Ws[:H], Ws[H:2 * H], Ws[2 * H:], bs.reshape(1, FEAT))

    return out
